# baseline (speedup 1.0000x reference)
import functools

import jax
import jax.numpy as jnp
from jax.experimental import pallas as pl
from jax.experimental.pallas import tpu as pltpu


def _lstm_kernel(sp_ref, vel_ref, wp_ref, bp_ref, wih_ref, whh_ref, bg_ref,
                 wo_ref, bo_ref, out_ref, gx_scr, h_scr, *, seq_len,
                 batch_blk, n_sp):
    T, Bc = seq_len, batch_blk
    H = whh_ref.shape[0]

    wih = wih_ref[...]
    wx = jnp.dot(wp_ref[...], wih, preferred_element_type=jnp.float32)
    bx = jnp.dot(bp_ref[...], wih, preferred_element_type=jnp.float32) \
        + bg_ref[...]

    sp_all = sp_ref[...].reshape(Bc * T, -1)
    vel_all = vel_ref[...].reshape(Bc * T, -1)
    gx = (jnp.dot(sp_all, wx[:n_sp, :],
                  preferred_element_type=jnp.float32)
          + jnp.dot(vel_all, wx[n_sp:, :],
                    preferred_element_type=jnp.float32)
          + bx)
    gx_scr[...] = gx.reshape(Bc, T, 4 * H)

    whh = whh_ref[...]
    h = jnp.zeros((Bc, H), jnp.float32)
    c = jnp.zeros((Bc, H), jnp.float32)
    for t in range(T):
        g = gx_scr[:, t, :] + jnp.dot(
            h, whh, preferred_element_type=jnp.float32)
        i_g = jax.nn.sigmoid(g[:, :H])
        f_g = jax.nn.sigmoid(g[:, H:2 * H])
        g_g = jnp.tanh(g[:, 2 * H:3 * H])
        o_g = jax.nn.sigmoid(g[:, 3 * H:])
        c = f_g * c + i_g * g_g
        h = o_g * jnp.tanh(c)
        h_scr[:, t, :] = h

    out_ref[...] = (jnp.dot(h_scr[...].reshape(Bc * T, H), wo_ref[...],
                            preferred_element_type=jnp.float32)
                    + bo_ref[...]).reshape(Bc, T, -1)


def kernel(spikes, velocities, wp_t, bp, wih_t, whh_t, bg, wo_t, bo):
    B, T, n_neurons = spikes.shape
    vel_dim = velocities.shape[2]
    D_in = wp_t.shape[0]
    H = wp_t.shape[1]
    n_out = wo_t.shape[1]
    n_fr_bins = n_out // n_neurons

    n_cores = 2 if B % 16 == 0 else 1
    Bc = B // n_cores

    kfn = functools.partial(_lstm_kernel, seq_len=T, batch_blk=Bc,
                            n_sp=n_neurons)

    out = pl.pallas_call(
        kfn,
        out_shape=jax.ShapeDtypeStruct((B, T, n_out), jnp.float32),
        grid=(n_cores,),
        in_specs=[
            pl.BlockSpec((Bc, T, n_neurons), lambda i: (i, 0, 0)),
            pl.BlockSpec((Bc, T, vel_dim), lambda i: (i, 0, 0)),
            pl.BlockSpec((D_in, H), lambda i: (0, 0)),
            pl.BlockSpec((1, H), lambda i: (0, 0)),
            pl.BlockSpec((H, 4 * H), lambda i: (0, 0)),
            pl.BlockSpec((H, 4 * H), lambda i: (0, 0)),
            pl.BlockSpec((1, 4 * H), lambda i: (0, 0)),
            pl.BlockSpec((H, n_out), lambda i: (0, 0)),
            pl.BlockSpec((1, n_out), lambda i: (0, 0)),
        ],
        out_specs=pl.BlockSpec((Bc, T, n_out), lambda i: (i, 0, 0)),
        scratch_shapes=[
            pltpu.VMEM((Bc, T, 4 * H), jnp.float32),
            pltpu.VMEM((Bc, T, H), jnp.float32),
        ],
        compiler_params=pltpu.CompilerParams(
            dimension_semantics=("parallel",)),
    )(spikes, velocities, wp_t, bp, wih_t, whh_t, bg, wo_t, bo)

    return out.reshape(B, T, n_neurons, n_fr_bins)


# speedup vs baseline: 1.2082x; 1.2082x over previous
import functools

import jax
import jax.numpy as jnp
from jax.experimental import pallas as pl
from jax.experimental.pallas import tpu as pltpu


def _lstm_kernel(sp_ref, vel_ref, wp_ref, bp_ref, wih_ref, whh_ref, bg_ref,
                 wo_ref, bo_ref, out_ref, gx_scr, gxt_scr, h_scr, *, seq_len,
                 batch_blk, n_sp):
    T, Bc = seq_len, batch_blk
    H = whh_ref.shape[0]

    wih = wih_ref[...]
    wx = jnp.dot(wp_ref[...], wih, preferred_element_type=jnp.float32)
    bx = jnp.dot(bp_ref[...], wih, preferred_element_type=jnp.float32) \
        + bg_ref[...]

    sp_all = sp_ref[...].reshape(Bc * T, -1)
    vel_all = vel_ref[...].reshape(Bc * T, -1)
    gx = (jnp.dot(sp_all, wx[:n_sp, :],
                  preferred_element_type=jnp.float32)
          + jnp.dot(vel_all, wx[n_sp:, :],
                    preferred_element_type=jnp.float32)
          + bx)
    gx_scr[...] = gx.reshape(Bc, T, 4 * H)

    whh = whh_ref[...]
    col = jax.lax.broadcasted_iota(jnp.int32, (1, 4 * H), 1)
    is_g = jnp.logical_and(col >= 2 * H, col < 3 * H)
    sc = jnp.where(is_g, 1.0, 0.5).astype(jnp.float32)
    off = jnp.where(is_g, 0.0, 0.5).astype(jnp.float32)

    for t in range(T):
        gxt_scr[t * Bc:(t + 1) * Bc, :] = gx_scr[:, t, :]

    n_ch = 8 if Bc % 64 == 0 else (4 if Bc % 32 == 0 else (2 if Bc % 16 == 0 else 1))
    Bh = Bc // n_ch
    hs = [jnp.zeros((Bh, H), jnp.float32) for _ in range(n_ch)]
    cs = [jnp.zeros((Bh, H), jnp.float32) for _ in range(n_ch)]
    for t in range(T):
        for k in range(n_ch):
            r0 = k * Bh
            g = gxt_scr[t * Bc + r0:t * Bc + r0 + Bh, :] + jnp.dot(
                hs[k], whh, preferred_element_type=jnp.float32)
            gates = jnp.tanh(g * sc) * sc + off
            i_g = gates[:, :H]
            f_g = gates[:, H:2 * H]
            g_g = gates[:, 2 * H:3 * H]
            o_g = gates[:, 3 * H:]
            cs[k] = f_g * cs[k] + i_g * g_g
            hs[k] = o_g * jnp.tanh(cs[k])
            h_scr[r0:r0 + Bh, t, :] = hs[k]

    out_ref[...] = (jnp.dot(h_scr[...].reshape(Bc * T, H), wo_ref[...],
                            preferred_element_type=jnp.float32)
                    + bo_ref[...]).reshape(Bc, T, -1)


def kernel(spikes, velocities, wp_t, bp, wih_t, whh_t, bg, wo_t, bo):
    B, T, n_neurons = spikes.shape
    vel_dim = velocities.shape[2]
    D_in = wp_t.shape[0]
    H = wp_t.shape[1]
    n_out = wo_t.shape[1]
    n_fr_bins = n_out // n_neurons

    n_cores = 2 if B % 16 == 0 else 1
    Bc = B // n_cores

    kfn = functools.partial(_lstm_kernel, seq_len=T, batch_blk=Bc,
                            n_sp=n_neurons)

    out = pl.pallas_call(
        kfn,
        out_shape=jax.ShapeDtypeStruct((B, T, n_out), jnp.float32),
        grid=(n_cores,),
        in_specs=[
            pl.BlockSpec((Bc, T, n_neurons), lambda i: (i, 0, 0)),
            pl.BlockSpec((Bc, T, vel_dim), lambda i: (i, 0, 0)),
            pl.BlockSpec((D_in, H), lambda i: (0, 0)),
            pl.BlockSpec((1, H), lambda i: (0, 0)),
            pl.BlockSpec((H, 4 * H), lambda i: (0, 0)),
            pl.BlockSpec((H, 4 * H), lambda i: (0, 0)),
            pl.BlockSpec((1, 4 * H), lambda i: (0, 0)),
            pl.BlockSpec((H, n_out), lambda i: (0, 0)),
            pl.BlockSpec((1, n_out), lambda i: (0, 0)),
        ],
        out_specs=pl.BlockSpec((Bc, T, n_out), lambda i: (i, 0, 0)),
        scratch_shapes=[
            pltpu.VMEM((Bc, T, 4 * H), jnp.float32),
            pltpu.VMEM((T * Bc, 4 * H), jnp.float32),
            pltpu.VMEM((Bc, T, H), jnp.float32),
        ],
        compiler_params=pltpu.CompilerParams(
            dimension_semantics=("parallel",)),
    )(spikes, velocities, wp_t, bp, wih_t, whh_t, bg, wo_t, bo)

    return out.reshape(B, T, n_neurons, n_fr_bins)


# speedup vs baseline: 1.3430x; 1.1116x over previous
import functools

import jax
import jax.numpy as jnp
from jax.experimental import pallas as pl
from jax.experimental.pallas import tpu as pltpu


def _lstm_kernel(sp_ref, vel_ref, wp_ref, bp_ref, wih_ref, whh_ref, bg_ref,
                 wo_ref, bo_ref, out_ref, gx_scr, gxt_scr, h_scr, *, seq_len,
                 batch_blk, n_sp):
    T, Bc = seq_len, batch_blk
    H = whh_ref.shape[0]

    wih = wih_ref[...]
    wx = jnp.dot(wp_ref[...], wih, preferred_element_type=jnp.float32)
    bx = jnp.dot(bp_ref[...], wih, preferred_element_type=jnp.float32) \
        + bg_ref[...]

    sp_all = sp_ref[...].reshape(Bc * T, -1)
    vel_all = vel_ref[...].reshape(Bc * T, -1)
    gx = (jnp.dot(sp_all, wx[:n_sp, :],
                  preferred_element_type=jnp.float32)
          + jnp.dot(vel_all, wx[n_sp:, :],
                    preferred_element_type=jnp.float32)
          + bx)
    gx_scr[...] = gx.reshape(Bc, T, 4 * H)

    whh = whh_ref[...]
    col = jax.lax.broadcasted_iota(jnp.int32, (1, 4 * H), 1)
    is_g = jnp.logical_and(col >= 2 * H, col < 3 * H)
    sc = jnp.where(is_g, 1.0, 0.5).astype(jnp.float32)
    off = jnp.where(is_g, 0.0, 0.5).astype(jnp.float32)

    for t in range(T):
        gxt_scr[t * Bc:(t + 1) * Bc, :] = gx_scr[:, t, :]

    n_ch = 4 if Bc % 32 == 0 else (2 if Bc % 16 == 0 else 1)
    Bh = Bc // n_ch
    hs = [jnp.zeros((Bh, H), jnp.float32) for _ in range(n_ch)]
    cs = [jnp.zeros((Bh, H), jnp.float32) for _ in range(n_ch)]
    for t in range(T):
        for k in range(n_ch):
            r0 = k * Bh
            g = gxt_scr[t * Bc + r0:t * Bc + r0 + Bh, :] + jnp.dot(
                hs[k], whh, preferred_element_type=jnp.float32)
            gates = jnp.tanh(g * sc) * sc + off
            i_g = gates[:, :H]
            f_g = gates[:, H:2 * H]
            g_g = gates[:, 2 * H:3 * H]
            o_g = gates[:, 3 * H:]
            cs[k] = f_g * cs[k] + i_g * g_g
            hs[k] = o_g * jnp.tanh(cs[k])
            h_scr[r0:r0 + Bh, t, :] = hs[k]

    out_ref[...] = (jnp.dot(h_scr[...].reshape(Bc * T, H), wo_ref[...],
                            preferred_element_type=jnp.float32)
                    + bo_ref[...]).reshape(Bc, T, -1)


def kernel(spikes, velocities, wp_t, bp, wih_t, whh_t, bg, wo_t, bo):
    B, T, n_neurons = spikes.shape
    vel_dim = velocities.shape[2]
    D_in = wp_t.shape[0]
    H = wp_t.shape[1]
    n_out = wo_t.shape[1]
    n_fr_bins = n_out // n_neurons

    n_cores = 2 if B % 16 == 0 else 1
    Bc = B // n_cores

    kfn = functools.partial(_lstm_kernel, seq_len=T, batch_blk=Bc,
                            n_sp=n_neurons)

    out = pl.pallas_call(
        kfn,
        out_shape=jax.ShapeDtypeStruct((B, T, n_out), jnp.float32),
        grid=(n_cores,),
        in_specs=[
            pl.BlockSpec((Bc, T, n_neurons), lambda i: (i, 0, 0)),
            pl.BlockSpec((Bc, T, vel_dim), lambda i: (i, 0, 0)),
            pl.BlockSpec((D_in, H), lambda i: (0, 0)),
            pl.BlockSpec((1, H), lambda i: (0, 0)),
            pl.BlockSpec((H, 4 * H), lambda i: (0, 0)),
            pl.BlockSpec((H, 4 * H), lambda i: (0, 0)),
            pl.BlockSpec((1, 4 * H), lambda i: (0, 0)),
            pl.BlockSpec((H, n_out), lambda i: (0, 0)),
            pl.BlockSpec((1, n_out), lambda i: (0, 0)),
        ],
        out_specs=pl.BlockSpec((Bc, T, n_out), lambda i: (i, 0, 0)),
        scratch_shapes=[
            pltpu.VMEM((Bc, T, 4 * H), jnp.float32),
            pltpu.VMEM((T * Bc, 4 * H), jnp.float32),
            pltpu.VMEM((Bc, T, H), jnp.float32),
        ],
        compiler_params=pltpu.CompilerParams(
            dimension_semantics=("parallel",)),
    )(spikes, velocities, wp_t, bp, wih_t, whh_t, bg, wo_t, bo)

    return out.reshape(B, T, n_neurons, n_fr_bins)
